# baseline (speedup 1.0000x reference)
import functools

import jax
import jax.numpy as jnp
from jax import lax
from jax.experimental import pallas as pl
from jax.experimental.pallas import tpu as pltpu
from jax.experimental.pallas import tpu_sc as plsc

EMBED = 64
NUM_CORES = 2
NUM_SUBCORES = 16
NUM_WORKERS = NUM_CORES * NUM_SUBCORES
CHUNK = 320


PACK_BLK = 6400
PACK_V1 = 524800
PACK_SHIFT = 480000


def _tc_pack_table(table_t):
    tc_mesh = pltpu.create_tensorcore_mesh("t", num_cores=2)

    @functools.partial(
        pl.kernel,
        out_type=jax.ShapeDtypeStruct((PACK_V1, 2 * EMBED), jnp.float32),
        mesh=tc_mesh,
    )
    def kfn(tt_hbm, out_hbm):
        def body(a_vmem, b_vmem, o_vmem):
            o_vmem[:, 0:EMBED] = a_vmem[...].T
            o_vmem[:, EMBED:2 * EMBED] = b_vmem[...].T

        pltpu.emit_pipeline(
            body,
            grid=(PACK_V1 // PACK_BLK,),
            in_specs=[
                pl.BlockSpec((EMBED, PACK_BLK), lambda g: (0, g)),
                pl.BlockSpec((EMBED, PACK_BLK),
                             lambda g: (0, g + PACK_SHIFT // PACK_BLK)),
            ],
            out_specs=[pl.BlockSpec((PACK_BLK, 2 * EMBED), lambda g: (g, 0))],
            core_axis_name="t",
            dimension_semantics=(pltpu.PARALLEL,),
        )(tt_hbm, tt_hbm, out_hbm)

    return kfn(table_t)


def _sc_gather(idx_row, packed_table, num_indices):
    per_w = num_indices // NUM_WORKERS
    n_chunks = per_w // CHUNK
    mesh = plsc.VectorSubcoreMesh(core_axis_name="c", subcore_axis_name="s")

    @functools.partial(
        pl.kernel,
        out_type=jax.ShapeDtypeStruct((num_indices, 2 * EMBED), jnp.float32),
        mesh=mesh,
        scratch_types=[
            pltpu.VMEM((per_w,), jnp.int32),
            pltpu.VMEM((CHUNK, 2 * EMBED), jnp.float32),
            pltpu.VMEM((CHUNK, 2 * EMBED), jnp.float32),
            pltpu.SemaphoreType.DMA,
            pltpu.SemaphoreType.DMA,
            pltpu.SemaphoreType.DMA,
            pltpu.SemaphoreType.DMA,
        ],
        compiler_params=pltpu.CompilerParams(use_tc_tiling_on_sc=False),
    )
    def kfn(tab_hbm, idx_hbm, out_hbm, idx_all, r0, r1, sg0, sg1, so0, so1):
        wid = lax.axis_index("s") * NUM_CORES + lax.axis_index("c")
        base = wid * per_w
        rows = (r0, r1)
        sg = (sg0, sg1)
        so = (so0, so1)

        pltpu.sync_copy(idx_hbm.at[pl.ds(base, per_w)], idx_all)

        def gather(j, b):
            return pltpu.make_async_copy(
                tab_hbm.at[idx_all.at[pl.ds(j * CHUNK, CHUNK)]],
                rows[b], sg[b])

        def writeback(j, b):
            return pltpu.make_async_copy(
                rows[b], out_hbm.at[pl.ds(base + j * CHUNK, CHUNK)], so[b])

        gather(0, 0).start()
        for j in range(1, n_chunks):
            b = j % 2
            if j >= 2:
                writeback(j - 2, b).wait()
            gather(j, b).start()
            gather(j - 1, 1 - b).wait()
            writeback(j - 1, 1 - b).start()
        last = n_chunks - 1
        gather(last, last % 2).wait()
        writeback(last, last % 2).start()
        writeback(last - 1, (last - 1) % 2).wait()
        writeback(last, last % 2).wait()

    return kfn(packed_table, idx_row)


def _tc_select_transpose(inter, x_t, batch, seq):
    def body(in_ref, x_ref, o_ref):
        s = pl.program_id(0)
        hi = x_ref[pl.ds(s, 1)][0] >= PACK_V1
        left = in_ref[:, 0:EMBED].T
        right = in_ref[:, EMBED:2 * EMBED].T
        o_ref[0] = jnp.where(hi[None, :], right, left)

    return pl.pallas_call(
        body,
        grid=(seq,),
        in_specs=[
            pl.BlockSpec((batch, 2 * EMBED), lambda s: (s, 0)),
            pl.BlockSpec((seq, batch), lambda s: (0, 0)),
        ],
        out_specs=pl.BlockSpec((1, EMBED, batch), lambda s: (s, 0, 0)),
        out_shape=jax.ShapeDtypeStruct((seq, EMBED, batch), jnp.float32),
        compiler_params=pltpu.CompilerParams(
            dimension_semantics=("parallel",)),
    )(inter, x_t)


def kernel(x, table):
    batch, seq = x.shape
    num_indices = batch * seq
    packed_table = _tc_pack_table(table.T)
    x_t = x.T.astype(jnp.int32)
    flat_t = x_t.reshape(num_indices)
    idx_row = jnp.where(flat_t < PACK_V1, flat_t, flat_t - PACK_SHIFT)
    inter = _sc_gather(idx_row, packed_table, num_indices)
    out = _tc_select_transpose(inter, x_t, batch, seq)
    return out.transpose(2, 0, 1)


# speedup vs baseline: 1.0001x; 1.0001x over previous
import functools

import jax
import jax.numpy as jnp
from jax import lax
from jax.experimental import pallas as pl
from jax.experimental.pallas import tpu as pltpu
from jax.experimental.pallas import tpu_sc as plsc

EMBED = 64
NUM_CORES = 2
NUM_SUBCORES = 16
NUM_WORKERS = NUM_CORES * NUM_SUBCORES
CHUNK = 320


PACK_BLK = 6400
PACK_V1 = 524800
PACK_SHIFT = 480000


def _tc_pack_table(table_t):
    tc_mesh = pltpu.create_tensorcore_mesh("t", num_cores=2)

    @functools.partial(
        pl.kernel,
        out_type=jax.ShapeDtypeStruct((PACK_V1, 2 * EMBED), jnp.float32),
        mesh=tc_mesh,
    )
    def kfn(tt_hbm, out_hbm):
        cid = lax.axis_index("t")
        n_blocks = PACK_V1 // PACK_BLK
        half_blocks = n_blocks // 2
        first = cid * half_blocks

        def body(a_vmem, b_vmem, o_vmem):
            o_vmem[:, 0:EMBED] = a_vmem[...].T
            o_vmem[:, EMBED:2 * EMBED] = b_vmem[...].T

        pltpu.emit_pipeline(
            body,
            grid=(half_blocks,),
            in_specs=[
                pl.BlockSpec((EMBED, PACK_BLK), lambda g: (0, first + g)),
                pl.BlockSpec(
                    (EMBED, PACK_BLK),
                    lambda g: (0, first + g + PACK_SHIFT // PACK_BLK)),
            ],
            out_specs=[pl.BlockSpec((PACK_BLK, 2 * EMBED),
                                    lambda g: (first + g, 0))],
        )(tt_hbm, tt_hbm, out_hbm)

    return kfn(table_t)


def _sc_gather(idx_row, packed_table, num_indices):
    per_w = num_indices // NUM_WORKERS
    n_chunks = per_w // CHUNK
    mesh = plsc.VectorSubcoreMesh(core_axis_name="c", subcore_axis_name="s")

    @functools.partial(
        pl.kernel,
        out_type=jax.ShapeDtypeStruct((num_indices, 2 * EMBED), jnp.float32),
        mesh=mesh,
        scratch_types=[
            pltpu.VMEM((per_w,), jnp.int32),
            pltpu.VMEM((CHUNK, 2 * EMBED), jnp.float32),
            pltpu.VMEM((CHUNK, 2 * EMBED), jnp.float32),
            pltpu.SemaphoreType.DMA,
            pltpu.SemaphoreType.DMA,
            pltpu.SemaphoreType.DMA,
            pltpu.SemaphoreType.DMA,
        ],
        compiler_params=pltpu.CompilerParams(use_tc_tiling_on_sc=False),
    )
    def kfn(tab_hbm, idx_hbm, out_hbm, idx_all, r0, r1, sg0, sg1, so0, so1):
        wid = lax.axis_index("s") * NUM_CORES + lax.axis_index("c")
        base = wid * per_w
        rows = (r0, r1)
        sg = (sg0, sg1)
        so = (so0, so1)

        pltpu.sync_copy(idx_hbm.at[pl.ds(base, per_w)], idx_all)

        def gather(j, b):
            return pltpu.make_async_copy(
                tab_hbm.at[idx_all.at[pl.ds(j * CHUNK, CHUNK)]],
                rows[b], sg[b])

        def writeback(j, b):
            return pltpu.make_async_copy(
                rows[b], out_hbm.at[pl.ds(base + j * CHUNK, CHUNK)], so[b])

        gather(0, 0).start()
        for j in range(1, n_chunks):
            b = j % 2
            if j >= 2:
                writeback(j - 2, b).wait()
            gather(j, b).start()
            gather(j - 1, 1 - b).wait()
            writeback(j - 1, 1 - b).start()
        last = n_chunks - 1
        gather(last, last % 2).wait()
        writeback(last, last % 2).start()
        writeback(last - 1, (last - 1) % 2).wait()
        writeback(last, last % 2).wait()

    return kfn(packed_table, idx_row)


def _tc_select_transpose(inter, x_t, batch, seq):
    def body(in_ref, x_ref, o_ref):
        s = pl.program_id(0)
        hi = x_ref[pl.ds(s, 1)][0] >= PACK_V1
        left = in_ref[:, 0:EMBED].T
        right = in_ref[:, EMBED:2 * EMBED].T
        o_ref[0] = jnp.where(hi[None, :], right, left)

    return pl.pallas_call(
        body,
        grid=(seq,),
        in_specs=[
            pl.BlockSpec((batch, 2 * EMBED), lambda s: (s, 0)),
            pl.BlockSpec((seq, batch), lambda s: (0, 0)),
        ],
        out_specs=pl.BlockSpec((1, EMBED, batch), lambda s: (s, 0, 0)),
        out_shape=jax.ShapeDtypeStruct((seq, EMBED, batch), jnp.float32),
        compiler_params=pltpu.CompilerParams(
            dimension_semantics=("parallel",)),
    )(inter, x_t)


def kernel(x, table):
    batch, seq = x.shape
    num_indices = batch * seq
    packed_table = _tc_pack_table(table.T)
    x_t = x.T.astype(jnp.int32)
    flat_t = x_t.reshape(num_indices)
    idx_row = jnp.where(flat_t < PACK_V1, flat_t, flat_t - PACK_SHIFT)
    inter = _sc_gather(idx_row, packed_table, num_indices)
    out = _tc_select_transpose(inter, x_t, batch, seq)
    return out.transpose(2, 0, 1)
